# Initial kernel scaffold; baseline (speedup 1.0000x reference)
import jax
import jax.numpy as jnp
from jax.experimental import pallas as pl


def kernel(x, edge_index, batch, enc_W, enc_b, eps, W1, b1, W2, b2, gamma, beta, cls_W, cls_b):
    raise NotImplementedError("write your pallas kernel here")


# speedup vs baseline: 5.5106x; 5.5106x over previous
import functools

import jax
import jax.numpy as jnp
from jax import lax
from jax.experimental import pallas as pl
from jax.experimental.pallas import tpu as pltpu
from jax.experimental.pallas import tpu_sc as plsc

N = 100000
E = 1600000
F = 128
H = 32
G = 64
L = 4
C = 2

NHALF = N // 2
SROWS = NHALF + 48
CH = 128
TILES = 16
NROW = 12544
EPAD = NROW * CH - E
RPT = NROW // TILES
IDXB = 56

def _sc_agg_body(h_hbm, src_hbm, dst_hbm, zeros_hbm, out_hbm,
                 srcb, dstr, dstl, rows, aggsh, sems):
    c = lax.axis_index("c")
    s = lax.axis_index("s")
    off = c * NHALF

    z0 = s * (SROWS // TILES)
    pltpu.sync_copy(zeros_hbm.at[pl.ds(z0, SROWS // TILES)],
                    aggsh.at[pl.ds(z0, SROWS // TILES)])
    plsc.subcore_barrier()

    @pl.loop(0, RPT // IDXB)
    def _ob(ob):
        r0 = s * RPT + ob * IDXB
        pltpu.sync_copy(src_hbm.at[pl.ds(r0, IDXB)], srcb)
        pltpu.sync_copy(dst_hbm.at[pl.ds(r0, IDXB)], dstr)

        @pl.loop(0, IDXB)
        def _loc(j):
            for jj in range(CH // 16):
                v = dstr[j, pl.ds(jj * 16, 16)]
                lv = v - off
                ok = (lv >= 0) & (lv < NHALF)
                dstl[j, pl.ds(jj * 16, 16)] = jnp.where(ok, lv, NHALF)

        pltpu.async_copy(h_hbm.at[srcb.at[0]], rows.at[0], sems.at[0])

        @pl.loop(0, IDXB)
        def _go(j):
            cb = j & 1
            nb = 1 - cb

            @pl.when(j < IDXB - 1)
            def _():
                pltpu.async_copy(h_hbm.at[srcb.at[j + 1]], rows.at[nb],
                                 sems.at[nb])

            pltpu.make_async_copy(h_hbm.at[srcb.at[j]], rows.at[cb],
                                  sems.at[cb]).wait()
            pltpu.sync_copy(rows.at[cb], aggsh.at[dstl.at[j]], add=True)

    plsc.subcore_barrier()
    w0 = s * (SROWS // TILES)
    pltpu.sync_copy(aggsh.at[pl.ds(w0, SROWS // TILES)],
                    out_hbm.at[c].at[pl.ds(w0, SROWS // TILES)])


_sc_agg_cached = None


def _sc_agg(h, src2, dst2, zeros):
    global _sc_agg_cached
    if _sc_agg_cached is None:
        mesh = plsc.VectorSubcoreMesh(core_axis_name="c",
                                      subcore_axis_name="s",
                                      num_cores=2, num_subcores=TILES)
        _sc_agg_cached = functools.partial(
            pl.kernel,
            out_type=jax.ShapeDtypeStruct((2, SROWS, H), jnp.float32),
            mesh=mesh,
            compiler_params=pltpu.CompilerParams(use_tc_tiling_on_sc=False),
            scratch_types=[
                pltpu.VMEM((IDXB, CH), jnp.int32),
                pltpu.VMEM((IDXB, CH), jnp.int32),
                pltpu.VMEM((IDXB, CH), jnp.int32),
                pltpu.VMEM((2, CH, H), jnp.float32),
                pltpu.VMEM_SHARED((SROWS, H), jnp.float32),
                pltpu.SemaphoreType.DMA((2,)),
            ],
        )(_sc_agg_body)
    return _sc_agg_cached(h, src2, dst2, zeros)


BN = 1000
NBLK = N // BN


def _enc_body(xr, wr, br, or_):
    or_[...] = jnp.dot(xr[...], wr[...],
                       preferred_element_type=jnp.float32) + br[...]


def _encode(x, enc_W, enc_b):
    return pl.pallas_call(
        _enc_body,
        grid=(NBLK,),
        in_specs=[pl.BlockSpec((BN, F), lambda i: (i, 0)),
                  pl.BlockSpec((F, H), lambda i: (0, 0)),
                  pl.BlockSpec((1, H), lambda i: (0, 0))],
        out_specs=pl.BlockSpec((BN, H), lambda i: (i, 0)),
        out_shape=jax.ShapeDtypeStruct((N, H), jnp.float32),
    )(x, enc_W, enc_b.reshape(1, H))


def _mlp_body(eps_ref, hr, ar, w1r, b1r, w2r, b2r, h2r, psr, pqr):
    i = pl.program_id(0)
    z = hr[...] * (1.0 + eps_ref[0]) + ar[...]
    u = jnp.maximum(
        jnp.dot(z, w1r[...], preferred_element_type=jnp.float32) + b1r[...],
        0.0)
    h2 = jnp.dot(u, w2r[...], preferred_element_type=jnp.float32) + b2r[...]
    h2r[...] = h2

    @pl.when(i == 0)
    def _():
        psr[...] = jnp.zeros_like(psr)
        pqr[...] = jnp.zeros_like(pqr)

    psr[...] += jnp.sum(h2, axis=0, keepdims=True)
    pqr[...] += jnp.sum(h2 * h2, axis=0, keepdims=True)


def _mlp(h, agg, eps_l, W1_l, b1_l, W2_l, b2_l):
    return pl.pallas_call(
        _mlp_body,
        grid=(NBLK,),
        in_specs=[pl.BlockSpec(memory_space=pltpu.SMEM),
                  pl.BlockSpec((BN, H), lambda i: (i, 0)),
                  pl.BlockSpec((BN, H), lambda i: (i, 0)),
                  pl.BlockSpec((H, H), lambda i: (0, 0)),
                  pl.BlockSpec((1, H), lambda i: (0, 0)),
                  pl.BlockSpec((H, H), lambda i: (0, 0)),
                  pl.BlockSpec((1, H), lambda i: (0, 0))],
        out_specs=[pl.BlockSpec((BN, H), lambda i: (i, 0)),
                   pl.BlockSpec((1, H), lambda i: (0, 0)),
                   pl.BlockSpec((1, H), lambda i: (0, 0))],
        out_shape=[jax.ShapeDtypeStruct((N, H), jnp.float32),
                   jax.ShapeDtypeStruct((1, H), jnp.float32),
                   jax.ShapeDtypeStruct((1, H), jnp.float32)],
    )(eps_l.reshape(1), h, agg, W1_l, b1_l.reshape(1, H), W2_l,
      b2_l.reshape(1, H))


def _norm_body(h2r, psr, pqr, gr, br, or_):
    mean = psr[...] * (1.0 / N)
    var = pqr[...] * (1.0 / N) - mean * mean
    scale = gr[...] * lax.rsqrt(var + 1e-5)
    or_[...] = jnp.maximum((h2r[...] - mean) * scale + br[...], 0.0)


def _norm(h2, ps, pq, gamma_l, beta_l):
    return pl.pallas_call(
        _norm_body,
        grid=(NBLK,),
        in_specs=[pl.BlockSpec((BN, H), lambda i: (i, 0)),
                  pl.BlockSpec((1, H), lambda i: (0, 0)),
                  pl.BlockSpec((1, H), lambda i: (0, 0)),
                  pl.BlockSpec((1, H), lambda i: (0, 0)),
                  pl.BlockSpec((1, H), lambda i: (0, 0))],
        out_specs=pl.BlockSpec((BN, H), lambda i: (i, 0)),
        out_shape=jax.ShapeDtypeStruct((N, H), jnp.float32),
    )(h2, ps, pq, gamma_l.reshape(1, H), beta_l.reshape(1, H))


def _pool_body(hr, br, wr, cbr, or_, acc):
    i = pl.program_id(0)

    @pl.when(i == 0)
    def _():
        acc[...] = jnp.zeros_like(acc)

    oh = (lax.broadcasted_iota(jnp.int32, (G, BN), 0) == br[0]).astype(
        jnp.float32)
    acc[...] += jnp.dot(oh, hr[...], preferred_element_type=jnp.float32)

    @pl.when(i == NBLK - 1)
    def _():
        or_[...] = jnp.dot(acc[...], wr[...],
                           preferred_element_type=jnp.float32) + cbr[...]


def _pool_cls(h, batch3, cls_W, cls_b):
    return pl.pallas_call(
        _pool_body,
        grid=(NBLK,),
        in_specs=[pl.BlockSpec((BN, H), lambda i: (i, 0)),
                  pl.BlockSpec((1, 1, BN), lambda i: (i, 0, 0)),
                  pl.BlockSpec((H, C), lambda i: (0, 0)),
                  pl.BlockSpec((1, C), lambda i: (0, 0))],
        out_specs=pl.BlockSpec((G, C), lambda i: (0, 0)),
        out_shape=jax.ShapeDtypeStruct((G, C), jnp.float32),
        scratch_shapes=[pltpu.VMEM((G, H), jnp.float32)],
    )(h, batch3, cls_W, cls_b.reshape(1, C))


def kernel(x, edge_index, batch, enc_W, enc_b, eps, W1, b1, W2, b2,
           gamma, beta, cls_W, cls_b):
    src2 = jnp.concatenate(
        [edge_index[0], jnp.zeros((EPAD,), jnp.int32)]).reshape(NROW, CH)
    dst2 = jnp.concatenate(
        [edge_index[1], jnp.full((EPAD,), N, jnp.int32)]).reshape(NROW, CH)
    zeros = jnp.zeros((SROWS, H), jnp.float32)
    batch3 = batch.astype(jnp.int32).reshape(NBLK, 1, BN)

    h = _encode(x, enc_W, enc_b)
    for l in range(L):
        o = _sc_agg(h, src2, dst2, zeros)
        agg = jnp.concatenate([o[0, :NHALF], o[1, :NHALF]], axis=0)
        h2, ps, pq = _mlp(h, agg, eps[l], W1[l], b1[l], W2[l], b2[l])
        h = _norm(h2, ps, pq, gamma[l], beta[l])
    return _pool_cls(h, batch3, cls_W, cls_b)
